# Initial kernel scaffold; baseline (speedup 1.0000x reference)
import functools

import jax
import jax.numpy as jnp
from jax import lax
from jax.experimental import pallas as pl
from jax.experimental.pallas import tpu as pltpu
from jax.experimental.pallas import tpu_sc as plsc

N_NODES = 100000
N_EDGES = 6400000
N_PAD = 100352
NC, NS = 2, 16
NW = NC * NS
EDGES_PER_W = N_EDGES // NW
CHUNK = 4000
N_CHUNKS = EDGES_PER_W // CHUNK
VEC = 16
SLICE = N_PAD // NS

TWO_PI = 6.283185307179586
PI = 3.141592653589793

_SIN_C = (0.99998456, -0.1666326, 0.0083123855, -0.00019316231, 2.173236e-06)
_COS_C = (0.99999946, -0.4999956, 0.04166103, -0.0013862747, 2.4253186e-05,
          -2.2193922e-07)


def _sin_poly(u):
    z = u * u
    p = _SIN_C[4]
    for c in (_SIN_C[3], _SIN_C[2], _SIN_C[1], _SIN_C[0]):
        p = p * z + c
    return p * u


def _cos_poly(u):
    z = u * u
    p = _COS_C[5]
    for c in (_COS_C[4], _COS_C[3], _COS_C[2], _COS_C[1], _COS_C[0]):
        p = p * z + c
    return p


_MESH = plsc.VectorSubcoreMesh(
    core_axis_name="c", subcore_axis_name="s", num_cores=NC, num_subcores=NS)


@functools.partial(
    pl.kernel,
    out_type=jax.ShapeDtypeStruct((NC, 2, N_PAD), jnp.float32),
    mesh=_MESH,
    scratch_types=[
        pltpu.VMEM_SHARED((N_PAD,), jnp.float32),
        pltpu.VMEM_SHARED((N_PAD,), jnp.float32),
        pltpu.VMEM_SHARED((N_PAD,), jnp.float32),
        pltpu.VMEM_SHARED((N_PAD,), jnp.float32),
        pltpu.VMEM((CHUNK,), jnp.int32),
        pltpu.VMEM((CHUNK,), jnp.int32),
        pltpu.VMEM((CHUNK, 4), jnp.float32),
        pltpu.VMEM((CHUNK,), jnp.float32),
        pltpu.VMEM((CHUNK,), jnp.float32),
        pltpu.VMEM((CHUNK,), jnp.float32),
        pltpu.VMEM((CHUNK,), jnp.float32),
        pltpu.VMEM((CHUNK,), jnp.float32),
        pltpu.VMEM((CHUNK,), jnp.float32),
        pltpu.VMEM((SLICE,), jnp.float32),
        pltpu.SemaphoreType.DMA,
    ],
)
def _edge_kernel(outT_hbm, edges_hbm, attr_hbm, acc_hbm,
                 th_sp, v_sp, acca_sp, accr_sp,
                 fidx, tidx, attr_v, thf, tht, vf, vt, act, rea, zbuf, gsem):
    c = lax.axis_index("c")
    s = lax.axis_index("s")
    w = s * NC + c
    off = s * SLICE

    pltpu.sync_copy(outT_hbm.at[2, pl.ds(off, SLICE)], th_sp.at[pl.ds(off, SLICE)])
    pltpu.sync_copy(outT_hbm.at[3, pl.ds(off, SLICE)], v_sp.at[pl.ds(off, SLICE)])
    zeros = jnp.zeros((VEC,), jnp.float32)

    def zero_body(i, carry):
        zbuf[pl.ds(i * VEC, VEC)] = zeros
        return carry

    lax.fori_loop(0, SLICE // VEC, zero_body, 0)
    pltpu.sync_copy(zbuf, acca_sp.at[pl.ds(off, SLICE)])
    pltpu.sync_copy(zbuf, accr_sp.at[pl.ds(off, SLICE)])
    plsc.subcore_barrier()

    base = w * EDGES_PER_W
    iot = lax.iota(jnp.int32, VEC)
    col0 = jnp.zeros((VEC,), jnp.int32)
    col1 = jnp.ones((VEC,), jnp.int32)

    def chunk_body(i, carry):
        eb = base + i * CHUNK
        pltpu.sync_copy(edges_hbm.at[0, pl.ds(eb, CHUNK)], fidx)
        pltpu.sync_copy(edges_hbm.at[1, pl.ds(eb, CHUNK)], tidx)
        pltpu.sync_copy(attr_hbm.at[pl.ds(eb, CHUNK), :], attr_v)
        d1 = pltpu.async_copy(th_sp.at[fidx], thf, gsem)
        d2 = pltpu.async_copy(th_sp.at[tidx], tht, gsem)
        d3 = pltpu.async_copy(v_sp.at[fidx], vf, gsem)
        d4 = pltpu.async_copy(v_sp.at[tidx], vt, gsem)
        d1.wait()
        d2.wait()
        d3.wait()
        d4.wait()

        def vec_body(j, carry2):
            sl = pl.ds(j * VEC, VEC)
            rows = iot + j * VEC
            g = plsc.load_gather(attr_v, [rows, col0])
            b = plsc.load_gather(attr_v, [rows, col1])
            d = jnp.abs(thf[sl] - tht[sl])
            u = lax.rem(d, TWO_PI) - PI
            sd = -_sin_poly(u)
            cd = -_cos_poly(u)
            av = jnp.abs(vf[sl]) * jnp.abs(vt[sl])
            act[sl] = av * (g * cd + b * sd)
            rea[sl] = av * (g * sd - b * cd)
            return carry2

        lax.fori_loop(0, CHUNK // VEC, vec_body, 0)
        pltpu.sync_copy(act, acca_sp.at[fidx], add=True)
        pltpu.sync_copy(rea, accr_sp.at[fidx], add=True)
        return carry

    lax.fori_loop(0, N_CHUNKS, chunk_body, 0)

    plsc.subcore_barrier()
    pltpu.sync_copy(acca_sp.at[pl.ds(off, SLICE)],
                    acc_hbm.at[c, 0, pl.ds(off, SLICE)])
    pltpu.sync_copy(accr_sp.at[pl.ds(off, SLICE)],
                    acc_hbm.at[c, 1, pl.ds(off, SLICE)])


def _loss_body(outT_ref, acc_ref, o_ref):
    act = acc_ref[0] + acc_ref[2]
    rea = acc_ref[1] + acc_ref[3]
    o0 = outT_ref[0]
    o1 = outT_ref[1]
    o_ref[0, 0] = jnp.sum(jnp.abs(o0 - act) + jnp.abs(o1 - rea))


_loss_call = pl.pallas_call(
    _loss_body,
    out_shape=jax.ShapeDtypeStruct((1, 1), jnp.float32),
    out_specs=pl.BlockSpec(memory_space=pltpu.SMEM),
)


def kernel(output, nodes, edges, attributes):
    edges = edges.astype(jnp.int32)
    outT = jnp.zeros((4, N_PAD), jnp.float32).at[:, :N_NODES].set(output.T)
    acc = _edge_kernel(outT, edges, attributes)
    acc4 = acc.reshape(4, 784, 128)
    loss = _loss_call(outT.reshape(4, 784, 128), acc4)
    return loss[0, 0]


# speedup vs baseline: 81.6947x; 81.6947x over previous
import functools

import jax
import jax.numpy as jnp
from jax import lax
from jax.experimental import pallas as pl
from jax.experimental.pallas import tpu as pltpu
from jax.experimental.pallas import tpu_sc as plsc

N_NODES = 100000
N_EDGES = 6400000
N_PAD = 100352
NC, NS = 2, 16
NW = NC * NS
EDGES_PER_W = N_EDGES // NW
CHUNK = 4000
N_CHUNKS = EDGES_PER_W // CHUNK
VEC = 16
SLICE = N_PAD // NS

TWO_PI = 6.283185307179586
PI = 3.141592653589793

_SIN_C = (0.99998456, -0.1666326, 0.0083123855, -0.00019316231, 2.173236e-06)
_COS_C = (0.99999946, -0.4999956, 0.04166103, -0.0013862747, 2.4253186e-05,
          -2.2193922e-07)


def _sin_poly(u):
    z = u * u
    p = _SIN_C[4]
    for c in (_SIN_C[3], _SIN_C[2], _SIN_C[1], _SIN_C[0]):
        p = p * z + c
    return p * u


def _cos_poly(u):
    z = u * u
    p = _COS_C[5]
    for c in (_COS_C[4], _COS_C[3], _COS_C[2], _COS_C[1], _COS_C[0]):
        p = p * z + c
    return p


_MESH = plsc.VectorSubcoreMesh(
    core_axis_name="c", subcore_axis_name="s", num_cores=NC, num_subcores=NS)


@functools.partial(
    pl.kernel,
    out_type=jax.ShapeDtypeStruct((NC * 2 * N_PAD,), jnp.float32),
    mesh=_MESH,
    scratch_types=[
        pltpu.VMEM_SHARED((N_PAD,), jnp.float32),
        pltpu.VMEM_SHARED((N_PAD,), jnp.float32),
        pltpu.VMEM_SHARED((N_PAD,), jnp.float32),
        pltpu.VMEM_SHARED((N_PAD,), jnp.float32),
        pltpu.VMEM((CHUNK,), jnp.int32),
        pltpu.VMEM((CHUNK,), jnp.int32),
        pltpu.VMEM((CHUNK,), jnp.float32),
        pltpu.VMEM((CHUNK,), jnp.float32),
        pltpu.VMEM((CHUNK,), jnp.float32),
        pltpu.VMEM((CHUNK,), jnp.float32),
        pltpu.VMEM((CHUNK,), jnp.float32),
        pltpu.VMEM((CHUNK,), jnp.float32),
        pltpu.VMEM((CHUNK,), jnp.float32),
        pltpu.VMEM((CHUNK,), jnp.float32),
        pltpu.VMEM((SLICE,), jnp.float32),
        pltpu.SemaphoreType.DMA,
    ],
)
def _edge_kernel(outT_hbm, edges_hbm, attr_hbm, acc_hbm,
                 th_sp, v_sp, acca_sp, accr_sp,
                 fidx, tidx, gbuf, bbuf, thf, tht, vf, vt, act, rea, zbuf,
                 gsem):
    c = lax.axis_index("c")
    s = lax.axis_index("s")
    w = s * NC + c
    off = s * SLICE

    pltpu.sync_copy(outT_hbm.at[pl.ds(2 * N_PAD + off, SLICE)],
                    th_sp.at[pl.ds(off, SLICE)])
    pltpu.sync_copy(outT_hbm.at[pl.ds(3 * N_PAD + off, SLICE)],
                    v_sp.at[pl.ds(off, SLICE)])
    zeros = jnp.zeros((VEC,), jnp.float32)

    def zero_body(i, carry):
        zbuf[pl.ds(i * VEC, VEC)] = zeros
        return carry

    lax.fori_loop(0, SLICE // VEC, zero_body, 0)
    pltpu.sync_copy(zbuf, acca_sp.at[pl.ds(off, SLICE)])
    pltpu.sync_copy(zbuf, accr_sp.at[pl.ds(off, SLICE)])
    plsc.subcore_barrier()

    base = w * EDGES_PER_W

    def chunk_body(i, carry):
        eb = base + i * CHUNK
        pltpu.sync_copy(edges_hbm.at[pl.ds(eb, CHUNK)], fidx)
        pltpu.sync_copy(edges_hbm.at[pl.ds(N_EDGES + eb, CHUNK)], tidx)
        pltpu.sync_copy(attr_hbm.at[pl.ds(eb, CHUNK)], gbuf)
        pltpu.sync_copy(attr_hbm.at[pl.ds(N_EDGES + eb, CHUNK)], bbuf)
        d1 = pltpu.async_copy(th_sp.at[fidx], thf, gsem)
        d2 = pltpu.async_copy(th_sp.at[tidx], tht, gsem)
        d3 = pltpu.async_copy(v_sp.at[fidx], vf, gsem)
        d4 = pltpu.async_copy(v_sp.at[tidx], vt, gsem)
        d1.wait()
        d2.wait()
        d3.wait()
        d4.wait()

        def vec_body(j, carry2):
            sl = pl.ds(j * VEC, VEC)
            g = gbuf[sl]
            b = bbuf[sl]
            d = jnp.abs(thf[sl] - tht[sl])
            u = lax.rem(d, TWO_PI) - PI
            sd = -_sin_poly(u)
            cd = -_cos_poly(u)
            av = jnp.abs(vf[sl]) * jnp.abs(vt[sl])
            act[sl] = av * (g * cd + b * sd)
            rea[sl] = av * (g * sd - b * cd)
            return carry2

        lax.fori_loop(0, CHUNK // VEC, vec_body, 0)
        pltpu.sync_copy(act, acca_sp.at[fidx], add=True)
        pltpu.sync_copy(rea, accr_sp.at[fidx], add=True)
        return carry

    lax.fori_loop(0, N_CHUNKS, chunk_body, 0)

    plsc.subcore_barrier()
    pltpu.sync_copy(acca_sp.at[pl.ds(off, SLICE)],
                    acc_hbm.at[pl.ds((c * 2 + 0) * N_PAD + off, SLICE)])
    pltpu.sync_copy(accr_sp.at[pl.ds(off, SLICE)],
                    acc_hbm.at[pl.ds((c * 2 + 1) * N_PAD + off, SLICE)])


def _loss_body(outT_ref, acc_ref, o_ref):
    act = acc_ref[0] + acc_ref[2]
    rea = acc_ref[1] + acc_ref[3]
    o0 = outT_ref[0]
    o1 = outT_ref[1]
    o_ref[0, 0] = jnp.sum(jnp.abs(o0 - act) + jnp.abs(o1 - rea))


_loss_call = pl.pallas_call(
    _loss_body,
    out_shape=jax.ShapeDtypeStruct((1, 1), jnp.float32),
    out_specs=pl.BlockSpec(memory_space=pltpu.SMEM),
)


def kernel(output, nodes, edges, attributes):
    edges = edges.astype(jnp.int32)
    outT = jnp.zeros((4, N_PAD), jnp.float32).at[:, :N_NODES].set(output.T)
    a01 = attributes[:, :2].T
    acc = _edge_kernel(outT.reshape(-1), edges.reshape(-1), a01.reshape(-1))
    acc4 = acc.reshape(4, 784, 128)
    loss = _loss_call(outT.reshape(4, 784, 128), acc4)
    return loss[0, 0]
